# baseline (speedup 1.0000x reference)
import functools

import jax
import jax.numpy as jnp
from jax import lax
from jax.experimental import pallas as pl
from jax.experimental.pallas import tpu as pltpu
from jax.experimental.pallas import tpu_sc as plsc

N_NODES = 10000
N_EDGES = 320000
H = 128
NC = 2
NS = 16
NW = NC * NS
PER_W = N_EDGES // NW
ROWS_PER_SUB = N_NODES // NS
W16 = 16

_MESH = plsc.VectorSubcoreMesh(
    core_axis_name="c", subcore_axis_name="s", num_cores=NC, num_subcores=NS)
_SC_PARAMS = pltpu.CompilerParams(use_tc_tiling_on_sc=False)


def _wid():
    return lax.axis_index("s") * NC + lax.axis_index("c")


_UR = 200
_NU = (N_EDGES // 8) // _UR


def _unit_range(wid):
    nu = jnp.where(wid < 8, 7, 6)
    start = 6 * wid + jnp.minimum(wid, 8)
    return start, nu


@functools.partial(
    pl.kernel,
    out_type=(jax.ShapeDtypeStruct((N_EDGES // 8, 128), jnp.float32),
              jax.ShapeDtypeStruct((N_EDGES // 8, 128), jnp.float32)),
    mesh=_MESH,
    scratch_types=[
        pltpu.VMEM((_UR,), jnp.int32),
        pltpu.VMEM((_UR,), jnp.int32),
        pltpu.VMEM((_UR, W16), jnp.float32),
        pltpu.VMEM((_UR, W16), jnp.float32),
        pltpu.SemaphoreType.DMA,
    ],
    compiler_params=_SC_PARAMS,
)
def _sc_gather_edges(tab, sendp8, recvp8, s_out, r_out, sidx, ridx, srows, rrows, sem):
    wid = _wid()
    start, nu = _unit_range(wid)

    def body(t, _):
        u = start + t
        rwb = u * _UR
        for j in range(8):
            pltpu.sync_copy(sendp8.at[j, u], sidx)
            pltpu.sync_copy(recvp8.at[j, u], ridx)
            pltpu.async_copy(tab.at[sidx], srows, sem).wait()
            pltpu.async_copy(tab.at[ridx], rrows, sem).wait()
            pltpu.sync_copy(srows, s_out.at[pl.ds(rwb, _UR), pl.ds(16 * j, 16)])
            pltpu.sync_copy(rrows, r_out.at[pl.ds(rwb, _UR), pl.ds(16 * j, 16)])
        return 0

    lax.fori_loop(0, nu, body, 0)


@functools.partial(
    pl.kernel,
    out_type=jax.ShapeDtypeStruct((NC, N_NODES, H), jnp.float32),
    mesh=_MESH,
    scratch_types=[
        pltpu.VMEM_SHARED((N_NODES, H), jnp.float32),
        pltpu.VMEM((_UR,), jnp.int32),
        pltpu.VMEM((_UR, H), jnp.float32),
    ],
    compiler_params=_SC_PARAMS,
)
def _sc_scatter_feat(feat, recvp8, zeros, agg_out, spmem, ridx, fbuf):
    c = lax.axis_index("c")
    s = lax.axis_index("s")
    wid = s * NC + c
    nb = s * ROWS_PER_SUB
    pltpu.sync_copy(zeros.at[pl.ds(nb, ROWS_PER_SUB)],
                    spmem.at[pl.ds(nb, ROWS_PER_SUB)])
    plsc.subcore_barrier()
    start, nu = _unit_range(wid)

    def body(t, _):
        u = start + t // 8
        j = t % 8
        rwb = u * _UR
        pltpu.sync_copy(recvp8.at[j, u], ridx)
        pltpu.sync_copy(feat.at[j, pl.ds(rwb, _UR)], fbuf)
        pltpu.sync_copy(fbuf, spmem.at[ridx], add=True)
        return 0

    lax.fori_loop(0, nu * 8, body, 0)
    plsc.subcore_barrier()
    pltpu.sync_copy(spmem.at[pl.ds(nb, ROWS_PER_SUB)],
                    agg_out.at[c, pl.ds(nb, ROWS_PER_SUB)])


@functools.partial(
    pl.kernel,
    out_type=jax.ShapeDtypeStruct((8, N_EDGES // 8, H), jnp.float32),
    mesh=_MESH,
    scratch_types=[
        pltpu.VMEM((_UR,), jnp.int32),
        pltpu.VMEM((_UR, H), jnp.float32),
        pltpu.SemaphoreType.DMA,
    ],
    compiler_params=_SC_PARAMS,
)
def _sc_gather_grad(gagg, recvp8, gfeat_out, ridx, fbuf, sem):
    wid = _wid()
    start, nu = _unit_range(wid)

    def body(t, _):
        u = start + t // 8
        j = t % 8
        rwb = u * _UR
        pltpu.sync_copy(recvp8.at[j, u], ridx)
        pltpu.async_copy(gagg.at[ridx], fbuf, sem).wait()
        pltpu.sync_copy(fbuf, gfeat_out.at[j, pl.ds(rwb, _UR)])
        return 0

    lax.fori_loop(0, nu * 8, body, 0)


@functools.partial(
    pl.kernel,
    out_type=jax.ShapeDtypeStruct((NC, 2, N_NODES, W16), jnp.float32),
    mesh=_MESH,
    scratch_types=[
        pltpu.VMEM_SHARED((N_NODES, W16), jnp.float32),
        pltpu.VMEM_SHARED((N_NODES, W16), jnp.float32),
        pltpu.VMEM((_UR,), jnp.int32),
        pltpu.VMEM((_UR,), jnp.int32),
        pltpu.VMEM((_UR, W16), jnp.float32),
    ],
    compiler_params=_SC_PARAMS,
)
def _sc_scatter_forces(gv, sendp8, recvp8, zeros, mf_out,
                       acc_r, acc_s, sidx, ridx, gbuf):
    c = lax.axis_index("c")
    s = lax.axis_index("s")
    wid = s * NC + c
    nb = s * ROWS_PER_SUB
    pltpu.sync_copy(zeros.at[pl.ds(nb, ROWS_PER_SUB)],
                    acc_r.at[pl.ds(nb, ROWS_PER_SUB)])
    pltpu.sync_copy(zeros.at[pl.ds(nb, ROWS_PER_SUB)],
                    acc_s.at[pl.ds(nb, ROWS_PER_SUB)])
    plsc.subcore_barrier()
    start, nu = _unit_range(wid)

    def body(t, _):
        u = start + t // 8
        j = t % 8
        rwb = u * _UR
        pltpu.sync_copy(recvp8.at[j, u], ridx)
        pltpu.sync_copy(sendp8.at[j, u], sidx)
        pltpu.sync_copy(gv.at[pl.ds(rwb, _UR), pl.ds(16 * j, 16)], gbuf)
        pltpu.sync_copy(gbuf, acc_r.at[ridx], add=True)
        pltpu.sync_copy(gbuf, acc_s.at[sidx], add=True)
        return 0

    lax.fori_loop(0, nu * 8, body, 0)
    plsc.subcore_barrier()
    pltpu.sync_copy(acc_r.at[pl.ds(nb, ROWS_PER_SUB)],
                    mf_out.at[c, 0, pl.ds(nb, ROWS_PER_SUB)])
    pltpu.sync_copy(acc_s.at[pl.ds(nb, ROWS_PER_SUB)],
                    mf_out.at[c, 1, pl.ds(nb, ROWS_PER_SUB)])


_BE = 6400
_BR = _BE // 8
_BN = 2000


def _edge_pre_packed(s_ref, r_ref, sh_ref, cellb_ref, w1b_ref, ehi_ref, elo_ref):
    s = s_ref[...]
    r = r_ref[...]
    vec = (r - s) + jnp.dot(sh_ref[...], cellb_ref[...],
                            preferred_element_type=jnp.float32)
    pre = jnp.dot(vec, w1b_ref[...], preferred_element_type=jnp.float32)
    ohs = s + r
    pre = pre + jnp.dot(ohs, ehi_ref[...], preferred_element_type=jnp.float32)
    pre = pre + jnp.dot(ohs, elo_ref[...], preferred_element_type=jnp.float32)
    return pre


def _tc_edge_fwd_body(s_ref, r_ref, sh_ref, cellb_ref, w1b_ref, ehi_ref,
                      elo_ref, feat_ref):
    pre = _edge_pre_packed(s_ref, r_ref, sh_ref, cellb_ref, w1b_ref,
                           ehi_ref, elo_ref)
    feat = pre * jax.nn.sigmoid(pre)
    for j in range(8):
        feat_ref[j] = feat[:, 128 * j:128 * (j + 1)]


def _tc_node_body(a0_ref, a1_ref, w2_ref, w2t_ref, w3_ref, esum_ref, gagg_ref):
    agg = a0_ref[...] + a1_ref[...]
    h = jnp.dot(agg, w2_ref[...], preferred_element_type=jnp.float32)
    sig = jax.nn.sigmoid(h)
    s = h * sig
    w3row = w3_ref[...]

    @pl.when(pl.program_id(0) == 0)
    def _():
        esum_ref[...] = jnp.zeros_like(esum_ref)

    s_bf = s.astype(jnp.bfloat16).astype(jnp.float32)
    w3_bf = w3row.astype(jnp.bfloat16).astype(jnp.float32)
    esum_ref[...] += jnp.reshape(jnp.sum(s_bf * w3_bf), (1, 1))
    g_h = w3row * (sig * (1.0 + h * (1.0 - sig)))
    gagg_ref[...] = jnp.dot(g_h, w2t_ref[...], preferred_element_type=jnp.float32)


def _tc_edge_bwd_body(gf_ref, s_ref, r_ref, sh_ref, cellb_ref, w1b_ref,
                      ehi_ref, elo_ref, w1tb_ref, gv_ref, ps_ref):
    pre = _edge_pre_packed(s_ref, r_ref, sh_ref, cellb_ref, w1b_ref,
                           ehi_ref, elo_ref)
    sig = jax.nn.sigmoid(pre)
    gf = jnp.concatenate([gf_ref[j] for j in range(8)], axis=1)
    gp = gf * (sig * (1.0 + pre * (1.0 - sig)))
    gv = jnp.dot(gp, w1tb_ref[...], preferred_element_type=jnp.float32)
    gv_ref[...] = gv

    @pl.when(pl.program_id(0) == 0)
    def _():
        ps_ref[...] = jnp.zeros_like(ps_ref)

    ps_ref[...] += lax.dot_general(sh_ref[...], gv, (((0,), (0,)), ((), ())),
                                   preferred_element_type=jnp.float32)


def _tc_final_body(r0_ref, r1_ref, s0_ref, s1_ref, pos_ref, forces_ref, sf_ref):
    mf = (r0_ref[...] + r1_ref[...]) - (s0_ref[...] + s1_ref[...])
    forces_ref[...] = -mf
    sf_ref[...] = lax.dot_general(mf, pos_ref[...], (((0,), (0,)), ((), ())),
                                  preferred_element_type=jnp.float32,
                                  precision=lax.Precision.HIGHEST)


def _diag_blocks(P, nb, a, b):
    return jnp.einsum('jujv->uv', P.reshape(nb, a, nb, b))


def kernel(positions, cell, shifts, senders, receivers, species, W1, emb, W2, w3):
    f32 = jnp.float32
    eye8 = jnp.eye(8, dtype=f32)
    node_tab = jnp.concatenate(
        [positions, jnp.zeros((N_NODES, 1), f32),
         jax.nn.one_hot(species, 10, dtype=f32),
         jnp.zeros((N_NODES, 2), f32)], axis=1)
    shp24 = jnp.reshape(shifts, (N_EDGES // 8, 24))
    w1p = jnp.pad(W1, ((0, W16 - 3), (0, 0)))
    w1blk = jnp.kron(eye8, w1p)
    w1tblk = jnp.kron(eye8, w1p.T)
    cellb = jnp.kron(eye8, jnp.pad(cell[0], ((0, 0), (0, W16 - 3))))
    embp = jnp.zeros((W16, H), f32).at[4:14].set(emb)
    emb_hi = embp.astype(jnp.bfloat16).astype(f32)
    ehiblk = jnp.kron(eye8, emb_hi)
    eloblk = jnp.kron(eye8, embp - emb_hi)
    w2t = W2.T
    w3row = w3[None, :]
    zeros_h = jnp.zeros((N_NODES, H), f32)
    zeros_w = jnp.zeros((N_NODES, W16), f32)
    pos_p = jnp.reshape(jnp.pad(positions, ((0, 0), (0, W16 - 3))),
                        (N_NODES // 8, 128))

    sendp8 = jnp.reshape(jnp.transpose(jnp.reshape(senders, (N_EDGES // 8, 8))),
                         (8, _NU, _UR))
    recvp8 = jnp.reshape(jnp.transpose(jnp.reshape(receivers, (N_EDGES // 8, 8))),
                         (8, _NU, _UR))
    s128, r128 = _sc_gather_edges(node_tab, sendp8, recvp8)

    ge = N_EDGES // _BE
    feat_p = pl.pallas_call(
        _tc_edge_fwd_body,
        grid=(ge,),
        in_specs=[
            pl.BlockSpec((_BR, 128), lambda i: (i, 0)),
            pl.BlockSpec((_BR, 128), lambda i: (i, 0)),
            pl.BlockSpec((_BR, 24), lambda i: (i, 0)),
            pl.BlockSpec((24, 128), lambda i: (0, 0)),
            pl.BlockSpec((128, 1024), lambda i: (0, 0)),
            pl.BlockSpec((128, 1024), lambda i: (0, 0)),
            pl.BlockSpec((128, 1024), lambda i: (0, 0)),
        ],
        out_specs=pl.BlockSpec((8, _BR, 128), lambda i: (0, i, 0)),
        out_shape=jax.ShapeDtypeStruct((8, N_EDGES // 8, 128), f32),
    )(s128, r128, shp24, cellb, w1blk, ehiblk, eloblk)

    agg2 = _sc_scatter_feat(feat_p, recvp8, zeros_h)

    gn = N_NODES // _BN
    esum, gagg = pl.pallas_call(
        _tc_node_body,
        grid=(gn,),
        in_specs=[
            pl.BlockSpec((_BN, H), lambda i: (i, 0)),
            pl.BlockSpec((_BN, H), lambda i: (i, 0)),
            pl.BlockSpec((H, H), lambda i: (0, 0)),
            pl.BlockSpec((H, H), lambda i: (0, 0)),
            pl.BlockSpec((1, H), lambda i: (0, 0)),
        ],
        out_specs=[
            pl.BlockSpec((1, 1), lambda i: (0, 0)),
            pl.BlockSpec((_BN, H), lambda i: (i, 0)),
        ],
        out_shape=[
            jax.ShapeDtypeStruct((1, 1), f32),
            jax.ShapeDtypeStruct((N_NODES, H), f32),
        ],
    )(agg2[0], agg2[1], W2, w2t, w3row)

    gf_p = _sc_gather_grad(gagg, recvp8)

    gv128, ps24 = pl.pallas_call(
        _tc_edge_bwd_body,
        grid=(ge,),
        in_specs=[
            pl.BlockSpec((8, _BR, 128), lambda i: (0, i, 0)),
            pl.BlockSpec((_BR, 128), lambda i: (i, 0)),
            pl.BlockSpec((_BR, 128), lambda i: (i, 0)),
            pl.BlockSpec((_BR, 24), lambda i: (i, 0)),
            pl.BlockSpec((24, 128), lambda i: (0, 0)),
            pl.BlockSpec((128, 1024), lambda i: (0, 0)),
            pl.BlockSpec((128, 1024), lambda i: (0, 0)),
            pl.BlockSpec((128, 1024), lambda i: (0, 0)),
            pl.BlockSpec((1024, 128), lambda i: (0, 0)),
        ],
        out_specs=[
            pl.BlockSpec((_BR, 128), lambda i: (i, 0)),
            pl.BlockSpec((24, 128), lambda i: (0, 0)),
        ],
        out_shape=[
            jax.ShapeDtypeStruct((N_EDGES // 8, 128), f32),
            jax.ShapeDtypeStruct((24, 128), f32),
        ],
    )(gf_p, s128, r128, shp24, cellb, w1blk, ehiblk, eloblk, w1tblk)

    mf4 = _sc_scatter_forces(gv128, sendp8, recvp8, zeros_w)
    mfp = jnp.reshape(mf4, (NC * 2, N_NODES // 8, 128))

    forces_p, sfull = pl.pallas_call(
        _tc_final_body,
        grid=(1,),
        in_specs=[
            pl.BlockSpec((N_NODES // 8, 128), lambda i: (0, 0)),
            pl.BlockSpec((N_NODES // 8, 128), lambda i: (0, 0)),
            pl.BlockSpec((N_NODES // 8, 128), lambda i: (0, 0)),
            pl.BlockSpec((N_NODES // 8, 128), lambda i: (0, 0)),
            pl.BlockSpec((N_NODES // 8, 128), lambda i: (0, 0)),
        ],
        out_specs=[
            pl.BlockSpec((N_NODES // 8, 128), lambda i: (0, 0)),
            pl.BlockSpec((128, 128), lambda i: (0, 0)),
        ],
        out_shape=[
            jax.ShapeDtypeStruct((N_NODES // 8, 128), f32),
            jax.ShapeDtypeStruct((128, 128), f32),
        ],
    )(mfp[0], mfp[2], mfp[1], mfp[3], pos_p)

    graph_energies = jnp.reshape(esum, (1,))
    forces = jnp.reshape(forces_p, (N_NODES, W16))[:, :3]
    pseudo_stress = _diag_blocks(ps24, 8, 3, W16)[:3, :3][None]
    stress_forces = _diag_blocks(sfull, 8, W16, W16)[:3, :3][None]
    det = jnp.linalg.det(cell)[:, None, None]
    det = jnp.where(det > 0.0, det, 1.0)
    stress_cell = jnp.transpose(pseudo_stress, (0, 2, 1)) @ cell
    viriel = stress_cell + stress_forces
    stress = -1.0 / det * viriel
    pressure = jnp.trace(stress, axis1=1, axis2=2)
    return (graph_energies, forces, stress,
            -1.0 / det * stress_cell, -1.0 / det * stress_forces, pressure)


# speedup vs baseline: 1.0768x; 1.0768x over previous
import functools

import jax
import jax.numpy as jnp
from jax import lax
from jax.experimental import pallas as pl
from jax.experimental.pallas import tpu as pltpu
from jax.experimental.pallas import tpu_sc as plsc

N_NODES = 10000
N_EDGES = 320000
H = 128
NC = 2
NS = 16
NW = NC * NS
PER_W = N_EDGES // NW
ROWS_PER_SUB = N_NODES // NS
W16 = 16

_MESH = plsc.VectorSubcoreMesh(
    core_axis_name="c", subcore_axis_name="s", num_cores=NC, num_subcores=NS)
_SC_PARAMS = pltpu.CompilerParams(use_tc_tiling_on_sc=False)


def _wid():
    return lax.axis_index("s") * NC + lax.axis_index("c")


_UR = 200
_NU = (N_EDGES // 8) // _UR


def _unit_range(wid):
    nu = jnp.where(wid < 8, 7, 6)
    start = 6 * wid + jnp.minimum(wid, 8)
    return start, nu


@functools.partial(
    pl.kernel,
    out_type=(jax.ShapeDtypeStruct((N_EDGES // 8, 128), jnp.float32),
              jax.ShapeDtypeStruct((N_EDGES // 8, 128), jnp.float32)),
    mesh=_MESH,
    scratch_types=[
        pltpu.VMEM((_UR,), jnp.int32),
        pltpu.VMEM((_UR,), jnp.int32),
        pltpu.VMEM((_UR, W16), jnp.float32),
        pltpu.VMEM((_UR, W16), jnp.float32),
        pltpu.SemaphoreType.DMA,
    ],
    compiler_params=_SC_PARAMS,
)
def _sc_gather_edges(tab, sendp8, recvp8, s_out, r_out, sidx, ridx, srows, rrows, sem):
    wid = _wid()
    start, nu = _unit_range(wid)

    def body(t, _):
        u = start + t
        rwb = u * _UR
        for j in range(8):
            pltpu.sync_copy(sendp8.at[j, u], sidx)
            pltpu.sync_copy(recvp8.at[j, u], ridx)
            pltpu.async_copy(tab.at[sidx], srows, sem).wait()
            pltpu.async_copy(tab.at[ridx], rrows, sem).wait()
            pltpu.sync_copy(srows, s_out.at[pl.ds(rwb, _UR), pl.ds(16 * j, 16)])
            pltpu.sync_copy(rrows, r_out.at[pl.ds(rwb, _UR), pl.ds(16 * j, 16)])
        return 0

    lax.fori_loop(0, nu, body, 0)


@functools.partial(
    pl.kernel,
    out_type=jax.ShapeDtypeStruct((NC, N_NODES, H), jnp.float32),
    mesh=_MESH,
    scratch_types=[
        pltpu.VMEM_SHARED((N_NODES, H), jnp.float32),
        pltpu.VMEM((_UR,), jnp.int32),
        pltpu.VMEM((_UR, H), jnp.float32),
    ],
    compiler_params=_SC_PARAMS,
)
def _sc_scatter_feat(feat, recvp8, zeros, agg_out, spmem, ridx, fbuf):
    c = lax.axis_index("c")
    s = lax.axis_index("s")
    wid = s * NC + c
    nb = s * ROWS_PER_SUB
    pltpu.sync_copy(zeros.at[pl.ds(nb, ROWS_PER_SUB)],
                    spmem.at[pl.ds(nb, ROWS_PER_SUB)])
    plsc.subcore_barrier()
    start, nu = _unit_range(wid)

    def body(t, _):
        u = start + t // 8
        j = t % 8
        rwb = u * _UR
        pltpu.sync_copy(recvp8.at[j, u], ridx)
        pltpu.sync_copy(feat.at[j, pl.ds(rwb, _UR)], fbuf)
        pltpu.sync_copy(fbuf, spmem.at[ridx], add=True)
        return 0

    lax.fori_loop(0, nu * 8, body, 0)
    plsc.subcore_barrier()
    pltpu.sync_copy(spmem.at[pl.ds(nb, ROWS_PER_SUB)],
                    agg_out.at[c, pl.ds(nb, ROWS_PER_SUB)])


@functools.partial(
    pl.kernel,
    out_type=jax.ShapeDtypeStruct((8, N_EDGES // 8, H), jnp.float32),
    mesh=_MESH,
    scratch_types=[
        pltpu.VMEM((_UR,), jnp.int32),
        pltpu.VMEM((_UR, H), jnp.float32),
        pltpu.SemaphoreType.DMA,
    ],
    compiler_params=_SC_PARAMS,
)
def _sc_gather_grad(gagg, recvp8, gfeat_out, ridx, fbuf, sem):
    wid = _wid()
    start, nu = _unit_range(wid)

    def body(t, _):
        u = start + t // 8
        j = t % 8
        rwb = u * _UR
        pltpu.sync_copy(recvp8.at[j, u], ridx)
        pltpu.async_copy(gagg.at[ridx], fbuf, sem).wait()
        pltpu.sync_copy(fbuf, gfeat_out.at[j, pl.ds(rwb, _UR)])
        return 0

    lax.fori_loop(0, nu * 8, body, 0)


_CG = 2000


@functools.partial(
    pl.kernel,
    out_type=jax.ShapeDtypeStruct((NC, 2, N_NODES, W16), jnp.float32),
    mesh=_MESH,
    scratch_types=[
        pltpu.VMEM_SHARED((N_NODES, W16), jnp.float32),
        pltpu.VMEM_SHARED((N_NODES, W16), jnp.float32),
        pltpu.VMEM((_CG,), jnp.int32),
        pltpu.VMEM((_CG,), jnp.int32),
        pltpu.VMEM((_CG, W16), jnp.float32),
    ],
    compiler_params=_SC_PARAMS,
)
def _sc_scatter_forces(gv, send, recv, zeros, mf_out,
                       acc_r, acc_s, sidx, ridx, gbuf):
    c = lax.axis_index("c")
    s = lax.axis_index("s")
    wid = s * NC + c
    nb = s * ROWS_PER_SUB
    pltpu.sync_copy(zeros.at[pl.ds(nb, ROWS_PER_SUB)],
                    acc_r.at[pl.ds(nb, ROWS_PER_SUB)])
    pltpu.sync_copy(zeros.at[pl.ds(nb, ROWS_PER_SUB)],
                    acc_s.at[pl.ds(nb, ROWS_PER_SUB)])
    plsc.subcore_barrier()
    for i in range(PER_W // _CG):
        base = wid * PER_W + i * _CG
        pltpu.sync_copy(recv.at[pl.ds(base, _CG)], ridx)
        pltpu.sync_copy(send.at[pl.ds(base, _CG)], sidx)
        pltpu.sync_copy(gv.at[pl.ds(base, _CG)], gbuf)
        pltpu.sync_copy(gbuf, acc_r.at[ridx], add=True)
        pltpu.sync_copy(gbuf, acc_s.at[sidx], add=True)
    plsc.subcore_barrier()
    pltpu.sync_copy(acc_r.at[pl.ds(nb, ROWS_PER_SUB)],
                    mf_out.at[c, 0, pl.ds(nb, ROWS_PER_SUB)])
    pltpu.sync_copy(acc_s.at[pl.ds(nb, ROWS_PER_SUB)],
                    mf_out.at[c, 1, pl.ds(nb, ROWS_PER_SUB)])


_BE = 6400
_BR = _BE // 8
_BN = 2000


def _edge_pre_packed(s_ref, r_ref, sh_ref, cellb_ref, w1b_ref, ehi_ref, elo_ref):
    s = s_ref[...]
    r = r_ref[...]
    vec = (r - s) + jnp.dot(sh_ref[...], cellb_ref[...],
                            preferred_element_type=jnp.float32)
    pre = jnp.dot(vec, w1b_ref[...], preferred_element_type=jnp.float32)
    ohs = s + r
    pre = pre + jnp.dot(ohs, ehi_ref[...], preferred_element_type=jnp.float32)
    pre = pre + jnp.dot(ohs, elo_ref[...], preferred_element_type=jnp.float32)
    return pre


def _tc_edge_fwd_body(s_ref, r_ref, sh_ref, cellb_ref, w1b_ref, ehi_ref,
                      elo_ref, feat_ref):
    pre = _edge_pre_packed(s_ref, r_ref, sh_ref, cellb_ref, w1b_ref,
                           ehi_ref, elo_ref)
    feat = pre * jax.nn.sigmoid(pre)
    for j in range(8):
        feat_ref[j] = feat[:, 128 * j:128 * (j + 1)]


def _tc_node_body(a0_ref, a1_ref, w2_ref, w2t_ref, w3_ref, esum_ref, gagg_ref):
    agg = a0_ref[...] + a1_ref[...]
    h = jnp.dot(agg, w2_ref[...], preferred_element_type=jnp.float32)
    sig = jax.nn.sigmoid(h)
    s = h * sig
    w3row = w3_ref[...]

    @pl.when(pl.program_id(0) == 0)
    def _():
        esum_ref[...] = jnp.zeros_like(esum_ref)

    s_bf = s.astype(jnp.bfloat16).astype(jnp.float32)
    w3_bf = w3row.astype(jnp.bfloat16).astype(jnp.float32)
    esum_ref[...] += jnp.reshape(jnp.sum(s_bf * w3_bf), (1, 1))
    g_h = w3row * (sig * (1.0 + h * (1.0 - sig)))
    gagg_ref[...] = jnp.dot(g_h, w2t_ref[...], preferred_element_type=jnp.float32)


def _tc_edge_bwd_body(gf_ref, s_ref, r_ref, sh_ref, cellb_ref, w1b_ref,
                      ehi_ref, elo_ref, w1tb_ref, gv_ref, ps_ref):
    pre = _edge_pre_packed(s_ref, r_ref, sh_ref, cellb_ref, w1b_ref,
                           ehi_ref, elo_ref)
    sig = jax.nn.sigmoid(pre)
    gf = jnp.concatenate([gf_ref[j] for j in range(8)], axis=1)
    gp = gf * (sig * (1.0 + pre * (1.0 - sig)))
    gv = jnp.dot(gp, w1tb_ref[...], preferred_element_type=jnp.float32)
    gv_ref[...] = gv

    @pl.when(pl.program_id(0) == 0)
    def _():
        ps_ref[...] = jnp.zeros_like(ps_ref)

    ps_ref[...] += lax.dot_general(sh_ref[...], gv, (((0,), (0,)), ((), ())),
                                   preferred_element_type=jnp.float32)


def _tc_final_body(r0_ref, r1_ref, s0_ref, s1_ref, pos_ref, forces_ref, sf_ref):
    mf = (r0_ref[...] + r1_ref[...]) - (s0_ref[...] + s1_ref[...])
    forces_ref[...] = -mf
    sf_ref[...] = lax.dot_general(mf, pos_ref[...], (((0,), (0,)), ((), ())),
                                  preferred_element_type=jnp.float32,
                                  precision=lax.Precision.HIGHEST)


def _diag_blocks(P, nb, a, b):
    return jnp.einsum('jujv->uv', P.reshape(nb, a, nb, b))


def kernel(positions, cell, shifts, senders, receivers, species, W1, emb, W2, w3):
    f32 = jnp.float32
    eye8 = jnp.eye(8, dtype=f32)
    node_tab = jnp.concatenate(
        [positions, jnp.zeros((N_NODES, 1), f32),
         jax.nn.one_hot(species, 10, dtype=f32),
         jnp.zeros((N_NODES, 2), f32)], axis=1)
    shp24 = jnp.reshape(shifts, (N_EDGES // 8, 24))
    w1p = jnp.pad(W1, ((0, W16 - 3), (0, 0)))
    w1blk = jnp.kron(eye8, w1p)
    w1tblk = jnp.kron(eye8, w1p.T)
    cellb = jnp.kron(eye8, jnp.pad(cell[0], ((0, 0), (0, W16 - 3))))
    embp = jnp.zeros((W16, H), f32).at[4:14].set(emb)
    emb_hi = embp.astype(jnp.bfloat16).astype(f32)
    ehiblk = jnp.kron(eye8, emb_hi)
    eloblk = jnp.kron(eye8, embp - emb_hi)
    w2t = W2.T
    w3row = w3[None, :]
    zeros_h = jnp.zeros((N_NODES, H), f32)
    zeros_w = jnp.zeros((N_NODES, W16), f32)
    pos_p = jnp.reshape(jnp.pad(positions, ((0, 0), (0, W16 - 3))),
                        (N_NODES // 8, 128))

    sendp8 = jnp.reshape(jnp.transpose(jnp.reshape(senders, (N_EDGES // 8, 8))),
                         (8, _NU, _UR))
    recvp8 = jnp.reshape(jnp.transpose(jnp.reshape(receivers, (N_EDGES // 8, 8))),
                         (8, _NU, _UR))
    s128, r128 = _sc_gather_edges(node_tab, sendp8, recvp8)

    ge = N_EDGES // _BE
    feat_p = pl.pallas_call(
        _tc_edge_fwd_body,
        grid=(ge,),
        in_specs=[
            pl.BlockSpec((_BR, 128), lambda i: (i, 0)),
            pl.BlockSpec((_BR, 128), lambda i: (i, 0)),
            pl.BlockSpec((_BR, 24), lambda i: (i, 0)),
            pl.BlockSpec((24, 128), lambda i: (0, 0)),
            pl.BlockSpec((128, 1024), lambda i: (0, 0)),
            pl.BlockSpec((128, 1024), lambda i: (0, 0)),
            pl.BlockSpec((128, 1024), lambda i: (0, 0)),
        ],
        out_specs=pl.BlockSpec((8, _BR, 128), lambda i: (0, i, 0)),
        out_shape=jax.ShapeDtypeStruct((8, N_EDGES // 8, 128), f32),
    )(s128, r128, shp24, cellb, w1blk, ehiblk, eloblk)

    agg2 = _sc_scatter_feat(feat_p, recvp8, zeros_h)

    gn = N_NODES // _BN
    esum, gagg = pl.pallas_call(
        _tc_node_body,
        grid=(gn,),
        in_specs=[
            pl.BlockSpec((_BN, H), lambda i: (i, 0)),
            pl.BlockSpec((_BN, H), lambda i: (i, 0)),
            pl.BlockSpec((H, H), lambda i: (0, 0)),
            pl.BlockSpec((H, H), lambda i: (0, 0)),
            pl.BlockSpec((1, H), lambda i: (0, 0)),
        ],
        out_specs=[
            pl.BlockSpec((1, 1), lambda i: (0, 0)),
            pl.BlockSpec((_BN, H), lambda i: (i, 0)),
        ],
        out_shape=[
            jax.ShapeDtypeStruct((1, 1), f32),
            jax.ShapeDtypeStruct((N_NODES, H), f32),
        ],
    )(agg2[0], agg2[1], W2, w2t, w3row)

    gf_p = _sc_gather_grad(gagg, recvp8)

    gv128, ps24 = pl.pallas_call(
        _tc_edge_bwd_body,
        grid=(ge,),
        in_specs=[
            pl.BlockSpec((8, _BR, 128), lambda i: (0, i, 0)),
            pl.BlockSpec((_BR, 128), lambda i: (i, 0)),
            pl.BlockSpec((_BR, 128), lambda i: (i, 0)),
            pl.BlockSpec((_BR, 24), lambda i: (i, 0)),
            pl.BlockSpec((24, 128), lambda i: (0, 0)),
            pl.BlockSpec((128, 1024), lambda i: (0, 0)),
            pl.BlockSpec((128, 1024), lambda i: (0, 0)),
            pl.BlockSpec((128, 1024), lambda i: (0, 0)),
            pl.BlockSpec((1024, 128), lambda i: (0, 0)),
        ],
        out_specs=[
            pl.BlockSpec((_BR, 128), lambda i: (i, 0)),
            pl.BlockSpec((24, 128), lambda i: (0, 0)),
        ],
        out_shape=[
            jax.ShapeDtypeStruct((N_EDGES // 8, 128), f32),
            jax.ShapeDtypeStruct((24, 128), f32),
        ],
    )(gf_p, s128, r128, shp24, cellb, w1blk, ehiblk, eloblk, w1tblk)

    gv16 = jnp.reshape(gv128, (N_EDGES, W16))
    mf4 = _sc_scatter_forces(gv16, senders, receivers, zeros_w)
    mfp = jnp.reshape(mf4, (NC * 2, N_NODES // 8, 128))

    forces_p, sfull = pl.pallas_call(
        _tc_final_body,
        grid=(1,),
        in_specs=[
            pl.BlockSpec((N_NODES // 8, 128), lambda i: (0, 0)),
            pl.BlockSpec((N_NODES // 8, 128), lambda i: (0, 0)),
            pl.BlockSpec((N_NODES // 8, 128), lambda i: (0, 0)),
            pl.BlockSpec((N_NODES // 8, 128), lambda i: (0, 0)),
            pl.BlockSpec((N_NODES // 8, 128), lambda i: (0, 0)),
        ],
        out_specs=[
            pl.BlockSpec((N_NODES // 8, 128), lambda i: (0, 0)),
            pl.BlockSpec((128, 128), lambda i: (0, 0)),
        ],
        out_shape=[
            jax.ShapeDtypeStruct((N_NODES // 8, 128), f32),
            jax.ShapeDtypeStruct((128, 128), f32),
        ],
    )(mfp[0], mfp[2], mfp[1], mfp[3], pos_p)

    graph_energies = jnp.reshape(esum, (1,))
    forces = jnp.reshape(forces_p, (N_NODES, W16))[:, :3]
    pseudo_stress = _diag_blocks(ps24, 8, 3, W16)[:3, :3][None]
    stress_forces = _diag_blocks(sfull, 8, W16, W16)[:3, :3][None]
    det = jnp.linalg.det(cell)[:, None, None]
    det = jnp.where(det > 0.0, det, 1.0)
    stress_cell = jnp.transpose(pseudo_stress, (0, 2, 1)) @ cell
    viriel = stress_cell + stress_forces
    stress = -1.0 / det * viriel
    pressure = jnp.trace(stress, axis1=1, axis2=2)
    return (graph_energies, forces, stress,
            -1.0 / det * stress_cell, -1.0 / det * stress_forces, pressure)
